# baseline (speedup 1.0000x reference)
import jax
import jax.numpy as jnp
from jax import lax
from jax.experimental import pallas as pl
from jax.experimental.pallas import tpu as pltpu
from jax.experimental.pallas import tpu_sc as plsc

N = 8192
D = 128
C = 100
CP = 112
NC = 2
NS = 16
NW = NC * NS
ROWS_ACC = N // NS
ROWS_OUT = N // NW
CLS_PER_TILE = CP // NS
LPR = 128


def _rsqrt_nr(x):
  bits = lax.bitcast_convert_type(x, jnp.int32)
  y = lax.bitcast_convert_type(
      jnp.int32(0x5F3759DF) - (bits >> 1), jnp.float32)
  for _ in range(3):
    t = x * y
    u = t * y
    y = y * (1.5 - 0.5 * u)
  return y


def _body(feat_hbm, lab_hbm, const_hbm, out_hbm,
          fv0, fv1, fv2, fv3, sqa, sqb, lab_v, ones_v,
          srow, qrow, ctmp, stdv,
          sem0, sem1, sem2, sem3, sem_c, sem_g, sem_s, sem_q,
          acc_s, acc_q, acc_c, std_s):
  s = lax.axis_index("s")
  c = lax.axis_index("c")
  gw = c * NS + s

  dg = pltpu.async_copy(lab_hbm.at[pl.ds(gw * 2, 2)],
                        lab_v.at[pl.ds(4, 2)], sem_g)
  do = pltpu.async_copy(const_hbm.at[pl.ds(0, LPR)], ones_v, sem_c)
  pltpu.sync_copy(lab_hbm.at[pl.ds(s * 4, 4)], lab_v.at[pl.ds(0, 4)])

  fbuf = (fv0, fv1, fv2, fv3)
  lsem = (sem0, sem1, sem2, sem3)
  loads = [pltpu.async_copy(
      feat_hbm.at[pl.ds(s * ROWS_ACC + t * LPR, LPR)], fbuf[t], lsem[t])
      for t in range(4)]

  @pl.when(s == 0)
  def _init():
    pltpu.sync_copy(const_hbm.at[pl.ds(LPR, CP)], sqa.at[pl.ds(0, CP)])
    z0 = pltpu.async_copy(sqa.at[pl.ds(0, CP)], acc_s, sem_s)
    z1 = pltpu.async_copy(sqa.at[pl.ds(0, CP)], acc_q, sem_s)
    z2 = pltpu.async_copy(sqa.at[pl.ds(0, CP)], acc_c, sem_s)
    z0.wait(); z1.wait(); z2.wait()

  do.wait()
  plsc.subcore_barrier()

  cns = [pltpu.async_copy(ones_v, acc_c.at[lab_v.at[j]], sem_c, add=True)
         for j in range(4)]

  def sqr(src, dst):
    def go(i, cy):
      for k in range(D // 16):
        v = src[i, pl.ds(k * 16, 16)]
        dst[i, pl.ds(k * 16, 16)] = v * v
      return cy
    lax.fori_loop(0, LPR, go, 0)

  qbuf = (sqa, sqb)
  ssc = [None] * 4
  qsc = [None] * 4
  for t in range(4):
    loads[t].wait()
    ssc[t] = pltpu.async_copy(fbuf[t], acc_s.at[lab_v.at[t]], sem_s,
                              add=True)
    if t >= 2:
      qsc[t - 2].wait()
    sqr(fbuf[t], qbuf[t % 2])
    qsc[t] = pltpu.async_copy(qbuf[t % 2], acc_q.at[lab_v.at[t]], sem_q,
                              add=True)

  for d in ssc + qsc[2:] + cns:
    d.wait()

  plsc.subcore_barrier()

  cls0 = s * CLS_PER_TILE
  f0 = pltpu.async_copy(acc_s.at[pl.ds(cls0, CLS_PER_TILE)], srow, sem0)
  f1 = pltpu.async_copy(acc_q.at[pl.ds(cls0, CLS_PER_TILE)], qrow, sem1)
  f2 = pltpu.async_copy(acc_c.at[pl.ds(cls0, CLS_PER_TILE)], ctmp, sem2)
  f0.wait(); f1.wait(); f2.wait()

  for r in range(CLS_PER_TILE):
    cnt = ctmp[r, pl.ds(0, 16)]
    inv = 1.0 / jnp.maximum(cnt, 1.0)
    for k in range(D // 16):
      sv = srow[r, pl.ds(k * 16, 16)]
      qv = qrow[r, pl.ds(k * 16, 16)]
      mean = sv * inv
      var = qv * inv - mean * mean
      var = jnp.maximum(var, 1e-30)
      stdv[r, pl.ds(k * 16, 16)] = var * _rsqrt_nr(var)

  pltpu.sync_copy(stdv, std_s.at[pl.ds(cls0, CLS_PER_TILE)])

  plsc.subcore_barrier()

  dg.wait()
  g0 = pltpu.async_copy(std_s.at[lab_v.at[4]], fv0, sem0)
  g1 = pltpu.async_copy(std_s.at[lab_v.at[5]], fv1, sem1)
  g0.wait()
  w0 = pltpu.async_copy(fv0, out_hbm.at[pl.ds(gw * ROWS_OUT, LPR)], sem_s)
  g1.wait()
  w1 = pltpu.async_copy(fv1, out_hbm.at[pl.ds(gw * ROWS_OUT + LPR, LPR)],
                        sem_q)
  w0.wait()
  w1.wait()


_sc_call = pl.kernel(
    _body,
    out_type=jax.ShapeDtypeStruct((N, D), jnp.float32),
    mesh=plsc.VectorSubcoreMesh(
        core_axis_name="c", subcore_axis_name="s",
        num_cores=NC, num_subcores=NS),
    scratch_types=[
        pltpu.VMEM((LPR, D), jnp.float32),
        pltpu.VMEM((LPR, D), jnp.float32),
        pltpu.VMEM((LPR, D), jnp.float32),
        pltpu.VMEM((LPR, D), jnp.float32),
        pltpu.VMEM((LPR, D), jnp.float32),
        pltpu.VMEM((LPR, D), jnp.float32),
        pltpu.VMEM((6, LPR), jnp.int32),
        pltpu.VMEM((LPR, D), jnp.float32),
        pltpu.VMEM((CLS_PER_TILE, D), jnp.float32),
        pltpu.VMEM((CLS_PER_TILE, D), jnp.float32),
        pltpu.VMEM((CLS_PER_TILE, D), jnp.float32),
        pltpu.VMEM((CLS_PER_TILE, D), jnp.float32),
        pltpu.SemaphoreType.DMA,
        pltpu.SemaphoreType.DMA,
        pltpu.SemaphoreType.DMA,
        pltpu.SemaphoreType.DMA,
        pltpu.SemaphoreType.DMA,
        pltpu.SemaphoreType.DMA,
        pltpu.SemaphoreType.DMA,
        pltpu.SemaphoreType.DMA,
        pltpu.VMEM_SHARED((CP, D), jnp.float32),
        pltpu.VMEM_SHARED((CP, D), jnp.float32),
        pltpu.VMEM_SHARED((CP, D), jnp.float32),
        pltpu.VMEM_SHARED((CP, D), jnp.float32),
    ],
)

_CONST = jnp.concatenate(
    [jnp.ones((LPR, D), jnp.float32), jnp.zeros((CP, D), jnp.float32)])


@jax.jit
def kernel(features, labels):
  lab2 = labels.astype(jnp.int32).reshape(N // LPR, LPR)
  return _sc_call(features, lab2, _CONST)


# speedup vs baseline: 1.0576x; 1.0576x over previous
import jax
import jax.numpy as jnp
from jax import lax
from jax.experimental import pallas as pl
from jax.experimental.pallas import tpu as pltpu
from jax.experimental.pallas import tpu_sc as plsc

N = 8192
D = 128
C = 100
CP = 112
NC = 2
NS = 16
NW = NC * NS
ROWS_ACC = N // NS
ROWS_OUT = N // NW
CLS_PER_TILE = CP // NS
LPR = 128


def _rsqrt_nr(x):
  bits = lax.bitcast_convert_type(x, jnp.int32)
  y = lax.bitcast_convert_type(
      jnp.int32(0x5F3759DF) - (bits >> 1), jnp.float32)
  for _ in range(3):
    t = x * y
    u = t * y
    y = y * (1.5 - 0.5 * u)
  return y


def _body(feat_hbm, lab_hbm, out_hbm,
          fva, fvb, sqa, sqb, lab_v, ones_v, srow, qrow, ctmp, stdv,
          sem_a, sem_b, sem_c, sem_g, sem_s, sem_q,
          acc_s, acc_q, acc_c, std_s):
  s = lax.axis_index("s")
  c = lax.axis_index("c")
  gw = c * NS + s

  zeros16 = jnp.zeros((16,), jnp.float32)
  ones16 = jnp.full((16,), 1.0, jnp.float32)

  dg = pltpu.async_copy(lab_hbm.at[pl.ds(gw * 2, 2)],
                        lab_v.at[pl.ds(4, 2)], sem_g)
  pltpu.sync_copy(lab_hbm.at[pl.ds(s * 4, 4)], lab_v.at[pl.ds(0, 4)])

  @pl.when(s == 0)
  def _init():
    def zf(i, cy):
      for k in range(D // 16):
        sqa[i, pl.ds(k * 16, 16)] = zeros16
      return cy
    lax.fori_loop(0, CP, zf, 0)
    pltpu.sync_copy(sqa.at[pl.ds(0, CP)], acc_s)
    pltpu.sync_copy(sqa.at[pl.ds(0, CP)], acc_q)
    pltpu.sync_copy(sqa.at[pl.ds(0, CP)], acc_c)

  def fill_ones(i, cy):
    for k in range(D // 16):
      ones_v[i, pl.ds(k * 16, 16)] = ones16
    return cy
  lax.fori_loop(0, LPR, fill_ones, 0)

  fbuf = (fva, fvb)
  qbuf = (sqa, sqb)
  lsem = (sem_a, sem_b)
  loads = [None] * 4
  loads[0] = pltpu.async_copy(feat_hbm.at[pl.ds(s * ROWS_ACC, LPR)],
                              fva, sem_a)
  loads[1] = pltpu.async_copy(feat_hbm.at[pl.ds(s * ROWS_ACC + LPR, LPR)],
                              fvb, sem_b)

  plsc.subcore_barrier()

  cns = [pltpu.async_copy(ones_v, acc_c.at[lab_v.at[j]], sem_c, add=True)
         for j in range(4)]

  def sqr(src, dst):
    def go(i, cy):
      for k in range(D // 16):
        v = src[i, pl.ds(k * 16, 16)]
        dst[i, pl.ds(k * 16, 16)] = v * v
      return cy
    lax.fori_loop(0, LPR, go, 0)

  ssc = [None] * 4
  qsc = [None] * 4
  for t in range(4):
    b = t % 2
    loads[t].wait()
    ssc[t] = pltpu.async_copy(fbuf[b], acc_s.at[lab_v.at[t]], sem_s,
                              add=True)
    if t >= 2:
      qsc[t - 2].wait()
    sqr(fbuf[b], qbuf[b])
    qsc[t] = pltpu.async_copy(qbuf[b], acc_q.at[lab_v.at[t]], sem_q,
                              add=True)
    if t < 2:
      ssc[t].wait()
      loads[t + 2] = pltpu.async_copy(
          feat_hbm.at[pl.ds(s * ROWS_ACC + (t + 2) * LPR, LPR)],
          fbuf[b], lsem[b])
    else:
      ssc[t].wait()

  for d in qsc[2:] + cns:
    d.wait()

  plsc.subcore_barrier()

  cls0 = s * CLS_PER_TILE
  f0 = pltpu.async_copy(acc_s.at[pl.ds(cls0, CLS_PER_TILE)], srow, sem_a)
  f1 = pltpu.async_copy(acc_q.at[pl.ds(cls0, CLS_PER_TILE)], qrow, sem_b)
  f2 = pltpu.async_copy(acc_c.at[pl.ds(cls0, CLS_PER_TILE)], ctmp, sem_c)
  f0.wait(); f1.wait(); f2.wait()

  for r in range(CLS_PER_TILE):
    cnt = ctmp[r, pl.ds(0, 16)]
    inv = 1.0 / jnp.maximum(cnt, 1.0)
    for k in range(D // 16):
      sv = srow[r, pl.ds(k * 16, 16)]
      qv = qrow[r, pl.ds(k * 16, 16)]
      mean = sv * inv
      var = qv * inv - mean * mean
      var = jnp.maximum(var, 1e-30)
      stdv[r, pl.ds(k * 16, 16)] = var * _rsqrt_nr(var)

  pltpu.sync_copy(stdv, std_s.at[pl.ds(cls0, CLS_PER_TILE)])

  plsc.subcore_barrier()

  dg.wait()
  g0 = pltpu.async_copy(std_s.at[lab_v.at[4]], fva, sem_a)
  g1 = pltpu.async_copy(std_s.at[lab_v.at[5]], fvb, sem_b)
  g0.wait()
  w0 = pltpu.async_copy(fva, out_hbm.at[pl.ds(gw * ROWS_OUT, LPR)], sem_s)
  g1.wait()
  w1 = pltpu.async_copy(fvb, out_hbm.at[pl.ds(gw * ROWS_OUT + LPR, LPR)],
                        sem_q)
  w0.wait()
  w1.wait()


_sc_call = pl.kernel(
    _body,
    out_type=jax.ShapeDtypeStruct((N, D), jnp.float32),
    mesh=plsc.VectorSubcoreMesh(
        core_axis_name="c", subcore_axis_name="s",
        num_cores=NC, num_subcores=NS),
    scratch_types=[
        pltpu.VMEM((LPR, D), jnp.float32),
        pltpu.VMEM((LPR, D), jnp.float32),
        pltpu.VMEM((LPR, D), jnp.float32),
        pltpu.VMEM((LPR, D), jnp.float32),
        pltpu.VMEM((6, LPR), jnp.int32),
        pltpu.VMEM((LPR, D), jnp.float32),
        pltpu.VMEM((CLS_PER_TILE, D), jnp.float32),
        pltpu.VMEM((CLS_PER_TILE, D), jnp.float32),
        pltpu.VMEM((CLS_PER_TILE, D), jnp.float32),
        pltpu.VMEM((CLS_PER_TILE, D), jnp.float32),
        pltpu.SemaphoreType.DMA,
        pltpu.SemaphoreType.DMA,
        pltpu.SemaphoreType.DMA,
        pltpu.SemaphoreType.DMA,
        pltpu.SemaphoreType.DMA,
        pltpu.SemaphoreType.DMA,
        pltpu.VMEM_SHARED((CP, D), jnp.float32),
        pltpu.VMEM_SHARED((CP, D), jnp.float32),
        pltpu.VMEM_SHARED((CP, D), jnp.float32),
        pltpu.VMEM_SHARED((CP, D), jnp.float32),
    ],
)


@jax.jit
def kernel(features, labels):
  lab2 = labels.astype(jnp.int32).reshape(N // LPR, LPR)
  return _sc_call(features, lab2)
